# baseline (speedup 1.0000x reference)
import functools
import math

import jax
import jax.numpy as jnp
from jax import lax
from jax.experimental import pallas as pl
from jax.experimental.pallas import tpu as pltpu
from jax.experimental.pallas import tpu_sc as plsc

D_MODEL = 1024
SCALE = math.sqrt(D_MODEL)
LANES = 16

NC = 2
NS = 16
NW = NC * NS

CHUNK = 32


def _make_gather_kernel(B):
    b_per_w = B // NW
    steps = b_per_w // CHUNK
    mesh = plsc.VectorSubcoreMesh(core_axis_name="c", subcore_axis_name="s")

    @functools.partial(
        pl.kernel,
        mesh=mesh,
        out_type=jax.ShapeDtypeStruct((B, D_MODEL), jnp.float32),
        scratch_types=[
            pltpu.VMEM((b_per_w,), jnp.int32),
            pltpu.VMEM((CHUNK, D_MODEL), jnp.float32),
            pltpu.SemaphoreType.DMA,
        ],
    )
    def gather_kernel(lut_hbm, idx_hbm, out_hbm, idx_v, rows_v, sem):
        wid = lax.axis_index("s") * NC + lax.axis_index("c")
        base = wid * b_per_w
        pltpu.sync_copy(idx_hbm.at[pl.ds(base, b_per_w)], idx_v)

        def step(g, carry):
            pltpu.async_copy(
                lut_hbm.at[idx_v.at[pl.ds(g * CHUNK, CHUNK)]], rows_v, sem
            ).wait()

            def scale_row(r, c):
                for j in range(D_MODEL // LANES):
                    sl = pl.ds(j * LANES, LANES)
                    rows_v[r, sl] = rows_v[r, sl] * SCALE
                return c

            lax.fori_loop(0, CHUNK, scale_row, 0)
            pltpu.sync_copy(rows_v, out_hbm.at[pl.ds(base + g * CHUNK, CHUNK)])
            return carry

        lax.fori_loop(0, steps, step, 0)

    return gather_kernel


def kernel(x, lut):
    B = x.shape[0] * x.shape[1]
    idx = x.reshape(B)
    out = _make_gather_kernel(B)(lut, idx)
    return out.reshape(x.shape[0], x.shape[1], D_MODEL)


# speedup vs baseline: 1.4659x; 1.4659x over previous
import functools
import math

import jax
import jax.numpy as jnp
from jax import lax
from jax.experimental import pallas as pl
from jax.experimental.pallas import tpu as pltpu
from jax.experimental.pallas import tpu_sc as plsc

D_MODEL = 1024
SCALE = math.sqrt(D_MODEL)
LANES = 16

NC = 2
NS = 16
NW = NC * NS

CHUNK = 16
NBUF = 3


def _make_gather_kernel(B):
    b_per_w = B // NW
    steps = b_per_w // CHUNK
    mesh = plsc.VectorSubcoreMesh(core_axis_name="c", subcore_axis_name="s")

    @functools.partial(
        pl.kernel,
        mesh=mesh,
        out_type=jax.ShapeDtypeStruct((B, D_MODEL), jnp.float32),
        scratch_types=(
            [pltpu.VMEM((b_per_w,), jnp.int32)]
            + [pltpu.VMEM((CHUNK, D_MODEL), jnp.float32) for _ in range(2 * NBUF)]
            + [pltpu.SemaphoreType.DMA for _ in range(2 * NBUF)]
        ),
    )
    def gather_kernel(lut_hbm, idx_hbm, out_hbm, idx_v, *bufs_and_sems):
        in_v = bufs_and_sems[:NBUF]
        out_v = bufs_and_sems[NBUF:2 * NBUF]
        gsem = bufs_and_sems[2 * NBUF:3 * NBUF]
        ssem = bufs_and_sems[3 * NBUF:4 * NBUF]

        wid = lax.axis_index("s") * NC + lax.axis_index("c")
        base = wid * b_per_w
        pltpu.sync_copy(idx_hbm.at[pl.ds(base, b_per_w)], idx_v)

        def start_gather(g, b):
            pltpu.async_copy(
                lut_hbm.at[idx_v.at[pl.ds(g * CHUNK, CHUNK)]], in_v[b], gsem[b]
            )

        def wait_gather(g, b):
            pltpu.make_async_copy(
                lut_hbm.at[idx_v.at[pl.ds(g * CHUNK, CHUNK)]], in_v[b], gsem[b]
            ).wait()

        def start_scatter(g, b):
            pltpu.async_copy(
                out_v[b], out_hbm.at[pl.ds(base + g * CHUNK, CHUNK)], ssem[b]
            )

        def wait_scatter(g, b):
            pltpu.make_async_copy(
                out_v[b], out_hbm.at[pl.ds(base + g * CHUNK, CHUNK)], ssem[b]
            ).wait()

        for b in range(NBUF):
            start_gather(b, b)

        for g in range(steps):
            b = g % NBUF
            wait_gather(g, b)
            if g >= NBUF:
                wait_scatter(g - NBUF, b)

            def scale_row(r, c, src=in_v[b], dst=out_v[b]):
                for j in range(D_MODEL // LANES):
                    sl = pl.ds(j * LANES, LANES)
                    dst[r, sl] = src[r, sl] * SCALE
                return c

            lax.fori_loop(0, CHUNK, scale_row, 0)
            start_scatter(g, b)
            if g + NBUF < steps:
                start_gather(g + NBUF, b)

        for g in range(steps - NBUF, steps):
            wait_scatter(g, g % NBUF)

    return gather_kernel


def kernel(x, lut):
    B = x.shape[0] * x.shape[1]
    idx = x.reshape(B)
    out = _make_gather_kernel(B)(lut, idx)
    return out.reshape(x.shape[0], x.shape[1], D_MODEL)


# speedup vs baseline: 1.4842x; 1.0125x over previous
import functools
import math

import jax
import jax.numpy as jnp
from jax import lax
from jax.experimental import pallas as pl
from jax.experimental.pallas import tpu as pltpu
from jax.experimental.pallas import tpu_sc as plsc

D_MODEL = 1024
SCALE = math.sqrt(D_MODEL)
LANES = 16

NC = 2
NS = 16
NW = NC * NS

CHUNK = 16
NBUF_IN = 4
NBUF_OUT = 3
LOOKAHEAD = NBUF_IN - 1


def _make_gather_kernel(B):
    b_per_w = B // NW
    steps = b_per_w // CHUNK
    mesh = plsc.VectorSubcoreMesh(core_axis_name="c", subcore_axis_name="s")

    @functools.partial(
        pl.kernel,
        mesh=mesh,
        out_type=jax.ShapeDtypeStruct((B, D_MODEL), jnp.float32),
        scratch_types=(
            [pltpu.VMEM((b_per_w,), jnp.int32)]
            + [pltpu.VMEM((CHUNK, D_MODEL), jnp.float32) for _ in range(NBUF_IN)]
            + [pltpu.VMEM((CHUNK, D_MODEL), jnp.float32) for _ in range(NBUF_OUT)]
            + [pltpu.SemaphoreType.DMA for _ in range(NBUF_IN + NBUF_OUT)]
        ),
    )
    def gather_kernel(lut_hbm, idx_hbm, out_hbm, idx_v, *bufs_and_sems):
        in_v = bufs_and_sems[:NBUF_IN]
        out_v = bufs_and_sems[NBUF_IN:NBUF_IN + NBUF_OUT]
        gsem = bufs_and_sems[NBUF_IN + NBUF_OUT:2 * NBUF_IN + NBUF_OUT]
        ssem = bufs_and_sems[2 * NBUF_IN + NBUF_OUT:]

        wid = lax.axis_index("s") * NC + lax.axis_index("c")
        base = wid * b_per_w
        pltpu.sync_copy(idx_hbm.at[pl.ds(base, b_per_w)], idx_v)

        def start_gather(g):
            b = g % NBUF_IN
            pltpu.async_copy(
                lut_hbm.at[idx_v.at[pl.ds(g * CHUNK, CHUNK)]], in_v[b], gsem[b]
            )

        def wait_gather(g):
            b = g % NBUF_IN
            pltpu.make_async_copy(
                lut_hbm.at[idx_v.at[pl.ds(g * CHUNK, CHUNK)]], in_v[b], gsem[b]
            ).wait()

        def start_scatter(g):
            b = g % NBUF_OUT
            pltpu.async_copy(
                out_v[b], out_hbm.at[pl.ds(base + g * CHUNK, CHUNK)], ssem[b]
            )

        def wait_scatter(g):
            b = g % NBUF_OUT
            pltpu.make_async_copy(
                out_v[b], out_hbm.at[pl.ds(base + g * CHUNK, CHUNK)], ssem[b]
            ).wait()

        for g in range(LOOKAHEAD):
            start_gather(g)

        for g in range(steps):
            wait_gather(g)
            if g + LOOKAHEAD < steps:
                start_gather(g + LOOKAHEAD)
            if g >= NBUF_OUT:
                wait_scatter(g - NBUF_OUT)

            def scale_row(r, c, src=in_v[g % NBUF_IN], dst=out_v[g % NBUF_OUT]):
                for j in range(D_MODEL // LANES):
                    sl = pl.ds(j * LANES, LANES)
                    dst[r, sl] = src[r, sl] * SCALE
                return c

            lax.fori_loop(0, CHUNK, scale_row, 0)
            start_scatter(g)

        for g in range(steps - NBUF_OUT, steps):
            wait_scatter(g)

    return gather_kernel


def kernel(x, lut):
    B = x.shape[0] * x.shape[1]
    idx = x.reshape(B)
    out = _make_gather_kernel(B)(lut, idx)
    return out.reshape(x.shape[0], x.shape[1], D_MODEL)
